# Initial kernel scaffold; baseline (speedup 1.0000x reference)
import jax
import jax.numpy as jnp
from jax.experimental import pallas as pl


def kernel(x, up_index, up_attr, face_index, face_attr, W_msg, b_msg, W1u, b1u, W2u, b2u, W1f, b1f, W2f, b2f, Wc, bc):
    raise NotImplementedError("write your pallas kernel here")


# speedup vs baseline: 2.5732x; 2.5732x over previous
import functools

import jax
import jax.numpy as jnp
from jax import lax
from jax.experimental import pallas as pl
from jax.experimental.pallas import tpu as pltpu
from jax.experimental.pallas import tpu_sc as plsc


def _mm_body(x_ref, w_ref, o_ref):
    o_ref[...] = jnp.dot(x_ref[...], w_ref[...], preferred_element_type=jnp.float32)


def _mm(x, w):
    n, d = x.shape
    return pl.pallas_call(
        _mm_body,
        out_shape=jax.ShapeDtypeStruct((n, w.shape[1]), jnp.float32),
    )(x, w)


def _mm_bias_body(x_ref, w_ref, b_ref, o_ref):
    o_ref[...] = (
        jnp.dot(x_ref[...], w_ref[...], preferred_element_type=jnp.float32)
        + b_ref[...]
    )


def _mm_bias(x, w, b, block):
    n, d = x.shape
    h = w.shape[1]
    grid = n // block
    return pl.pallas_call(
        _mm_bias_body,
        grid=(grid,),
        in_specs=[
            pl.BlockSpec((block, d), lambda i: (i, 0)),
            pl.BlockSpec((d, h), lambda i: (0, 0)),
            pl.BlockSpec((1, h), lambda i: (0, 0)),
        ],
        out_specs=pl.BlockSpec((block, h), lambda i: (i, 0)),
        out_shape=jax.ShapeDtypeStruct((n, h), jnp.float32),
    )(x, w, b.reshape(1, -1))


def _mlp_body(au_ref, af_ref, w1u_ref, b1u_ref, w2u_ref, b2u_ref,
              w1f_ref, b1f_ref, w2f_ref, b2f_ref, wc1_ref, wc2_ref, bc_ref,
              o_ref):
    f32 = jnp.float32
    hu = jnp.maximum(
        jnp.dot(au_ref[...], w1u_ref[...], preferred_element_type=f32)
        + b1u_ref[...], 0.0)
    ou = jnp.maximum(
        jnp.dot(hu, w2u_ref[...], preferred_element_type=f32) + b2u_ref[...],
        0.0)
    hf = jnp.maximum(
        jnp.dot(af_ref[...], w1f_ref[...], preferred_element_type=f32)
        + b1f_ref[...], 0.0)
    of = jnp.maximum(
        jnp.dot(hf, w2f_ref[...], preferred_element_type=f32) + b2f_ref[...],
        0.0)
    o_ref[...] = jnp.maximum(
        jnp.dot(ou, wc1_ref[...], preferred_element_type=f32)
        + jnp.dot(of, wc2_ref[...], preferred_element_type=f32)
        + bc_ref[...], 0.0)


def _mlps(acc_up, acc_f, W1u, b1u, W2u, b2u, W1f, b1f, W2f, b2f, Wc, bc,
          block):
    n, d = acc_up.shape
    h = W1u.shape[1]
    grid = n // block
    wspec = pl.BlockSpec((d, h), lambda i: (0, 0))
    bspec = pl.BlockSpec((1, h), lambda i: (0, 0))
    nspec = pl.BlockSpec((block, d), lambda i: (i, 0))
    return pl.pallas_call(
        _mlp_body,
        grid=(grid,),
        in_specs=[nspec, nspec,
                  wspec, bspec, wspec, bspec,
                  wspec, bspec, wspec, bspec,
                  wspec, wspec, bspec],
        out_specs=pl.BlockSpec((block, h), lambda i: (i, 0)),
        out_shape=jax.ShapeDtypeStruct((n, h), jnp.float32),
    )(acc_up, acc_f,
      W1u, b1u.reshape(1, -1), W2u, b2u.reshape(1, -1),
      W1f, b1f.reshape(1, -1), W2f, b2f.reshape(1, -1),
      Wc[:d], Wc[d:], bc.reshape(1, -1))


_CH = 80


def _sc_task(table, aw, x, src3, dst3):
    n, d = x.shape
    e = src3.shape[0] * src3.shape[1] * src3.shape[2]
    info = plsc.get_sparse_core_info()
    ns = info.num_subcores
    ept = e // ns
    nch = ept // _CH
    half = n // 2
    trash = half
    rpt = (half // ns) // 8 * 8
    tail = half - ns * rpt

    mesh = plsc.VectorSubcoreMesh(core_axis_name="c", subcore_axis_name="s")
    have_aw = aw is not None

    scratch = [
        pltpu.VMEM((nch, _CH), jnp.int32),
        pltpu.VMEM((nch, _CH), jnp.int32),
        pltpu.VMEM((_CH, d), jnp.float32),
        pltpu.VMEM((_CH, d), jnp.float32),
        pltpu.VMEM_SHARED((half + 8, d), jnp.float32),
        pltpu.SemaphoreType.DMA,
        pltpu.SemaphoreType.DMA,
    ]

    @functools.partial(
        pl.kernel,
        mesh=mesh,
        out_type=jax.ShapeDtypeStruct((n, d), jnp.float32),
        scratch_types=scratch,
    )
    def k(table_hbm, aw_hbm, src_hbm, dst_hbm, x_hbm,
          out, srcb, dstb, gbuf, abuf, acc, gsem, asem):
        cid = lax.axis_index("c")
        sid = lax.axis_index("s")
        lo = cid * half
        r0 = sid * rpt

        pltpu.sync_copy(x_hbm.at[pl.ds(lo + r0, rpt)], acc.at[pl.ds(r0, rpt)])

        @pl.when(sid == ns - 1)
        def _init_tail():
            t = ns * rpt
            pltpu.sync_copy(x_hbm.at[pl.ds(lo + t, tail)],
                            acc.at[pl.ds(t, tail)])

        pltpu.sync_copy(src_hbm.at[sid], srcb)
        pltpu.sync_copy(dst_hbm.at[sid], dstb)

        @plsc.parallel_loop(0, nch, 1, unroll=2)
        def _clamp(c):
            for kk in range(_CH // 16):
                s = pl.ds(kk * 16, 16)
                dv = dstb[c, s]
                keep = (dv >= lo) & (dv < lo + half)
                dstb[c, s] = jnp.where(keep, dv - lo, trash)

        plsc.subcore_barrier()

        ebase = sid * ept

        @pl.loop(0, nch)
        def _(c):
            gd = pltpu.async_copy(table_hbm.at[srcb.at[c]], gbuf, gsem)
            if have_aw:
                ad = pltpu.async_copy(
                    aw_hbm.at[pl.ds(ebase + c * _CH, _CH)], abuf, asem)
            gd.wait()
            if have_aw:
                ad.wait()

                @plsc.parallel_loop(0, _CH, 1, unroll=2)
                def _(r):
                    for kk in range(d // 16):
                        s = pl.ds(kk * 16, 16)
                        abuf[r, s] = jnp.maximum(abuf[r, s] + gbuf[r, s], 0.0)

                pltpu.sync_copy(abuf, acc.at[dstb.at[c]], add=True)
            else:
                pltpu.sync_copy(gbuf, acc.at[dstb.at[c]], add=True)

        plsc.subcore_barrier()

        pltpu.sync_copy(acc.at[pl.ds(r0, rpt)], out.at[pl.ds(lo + r0, rpt)])

        @pl.when(sid == ns - 1)
        def _flush_tail():
            t = ns * rpt
            pltpu.sync_copy(acc.at[pl.ds(t, tail)],
                            out.at[pl.ds(lo + t, tail)])

    return k(table, aw if have_aw else table, src3, dst3, x)


def kernel(x, up_index, up_attr, face_index, face_attr,
           W_msg, b_msg, W1u, b1u, W2u, b2u, W1f, b1f, W2f, b2f, Wc, bc):
    n, d = x.shape
    e = up_attr.shape[0]

    xw = _mm(x, W_msg[:d])
    aw = _mm_bias(up_attr, W_msg[d:], b_msg, block=2000)

    ns = plsc.get_sparse_core_info().num_subcores
    idx_shape = (ns, e // ns // _CH, _CH)
    usrc3 = up_index[0].reshape(idx_shape)
    udst3 = up_index[1].reshape(idx_shape)
    fsrc3 = face_index[0].reshape(idx_shape)
    fdst3 = face_index[1].reshape(idx_shape)

    acc_up = _sc_task(xw, aw, x, usrc3, udst3)
    acc_f = _sc_task(face_attr, None, x, fsrc3, fdst3)

    return _mlps(acc_up, acc_f, W1u, b1u, W2u, b2u, W1f, b1f, W2f, b2f,
                 Wc, bc, block=1000)


# speedup vs baseline: 2.5745x; 1.0005x over previous
import functools

import jax
import jax.numpy as jnp
from jax import lax
from jax.experimental import pallas as pl
from jax.experimental.pallas import tpu as pltpu
from jax.experimental.pallas import tpu_sc as plsc


def _mm_body(x_ref, w_ref, o_ref):
    o_ref[...] = jnp.dot(x_ref[...], w_ref[...], preferred_element_type=jnp.float32)


def _mm(x, w):
    n, d = x.shape
    return pl.pallas_call(
        _mm_body,
        out_shape=jax.ShapeDtypeStruct((n, w.shape[1]), jnp.float32),
    )(x, w)


def _mm_bias_body(x_ref, w_ref, b_ref, o_ref):
    o_ref[...] = (
        jnp.dot(x_ref[...], w_ref[...], preferred_element_type=jnp.float32)
        + b_ref[...]
    )


def _mm_bias(x, w, b, block):
    n, d = x.shape
    h = w.shape[1]
    grid = n // block
    return pl.pallas_call(
        _mm_bias_body,
        grid=(grid,),
        in_specs=[
            pl.BlockSpec((block, d), lambda i: (i, 0)),
            pl.BlockSpec((d, h), lambda i: (0, 0)),
            pl.BlockSpec((1, h), lambda i: (0, 0)),
        ],
        out_specs=pl.BlockSpec((block, h), lambda i: (i, 0)),
        out_shape=jax.ShapeDtypeStruct((n, h), jnp.float32),
    )(x, w, b.reshape(1, -1))


def _mlp_body(au_ref, af_ref, w1u_ref, b1u_ref, w2u_ref, b2u_ref,
              w1f_ref, b1f_ref, w2f_ref, b2f_ref, wc1_ref, wc2_ref, bc_ref,
              o_ref):
    f32 = jnp.float32
    hu = jnp.maximum(
        jnp.dot(au_ref[...], w1u_ref[...], preferred_element_type=f32)
        + b1u_ref[...], 0.0)
    ou = jnp.maximum(
        jnp.dot(hu, w2u_ref[...], preferred_element_type=f32) + b2u_ref[...],
        0.0)
    hf = jnp.maximum(
        jnp.dot(af_ref[...], w1f_ref[...], preferred_element_type=f32)
        + b1f_ref[...], 0.0)
    of = jnp.maximum(
        jnp.dot(hf, w2f_ref[...], preferred_element_type=f32) + b2f_ref[...],
        0.0)
    o_ref[...] = jnp.maximum(
        jnp.dot(ou, wc1_ref[...], preferred_element_type=f32)
        + jnp.dot(of, wc2_ref[...], preferred_element_type=f32)
        + bc_ref[...], 0.0)


def _mlps(acc_up, acc_f, W1u, b1u, W2u, b2u, W1f, b1f, W2f, b2f, Wc, bc,
          block):
    n, d = acc_up.shape
    h = W1u.shape[1]
    grid = n // block
    wspec = pl.BlockSpec((d, h), lambda i: (0, 0))
    bspec = pl.BlockSpec((1, h), lambda i: (0, 0))
    nspec = pl.BlockSpec((block, d), lambda i: (i, 0))
    return pl.pallas_call(
        _mlp_body,
        grid=(grid,),
        in_specs=[nspec, nspec,
                  wspec, bspec, wspec, bspec,
                  wspec, bspec, wspec, bspec,
                  wspec, wspec, bspec],
        out_specs=pl.BlockSpec((block, h), lambda i: (i, 0)),
        out_shape=jax.ShapeDtypeStruct((n, h), jnp.float32),
    )(acc_up, acc_f,
      W1u, b1u.reshape(1, -1), W2u, b2u.reshape(1, -1),
      W1f, b1f.reshape(1, -1), W2f, b2f.reshape(1, -1),
      Wc[:d], Wc[d:], bc.reshape(1, -1))


_CH = 80


def _sc_task(table, aw, x, src3, dst3):
    n, d = x.shape
    e = src3.shape[0] * src3.shape[1] * src3.shape[2]
    info = plsc.get_sparse_core_info()
    ns = info.num_subcores
    ept = e // ns
    nch = ept // _CH
    half = n // 2
    trash = half
    rpt = (half // ns) // 8 * 8
    tail = half - ns * rpt

    mesh = plsc.VectorSubcoreMesh(core_axis_name="c", subcore_axis_name="s")
    have_aw = aw is not None

    buf_shape = pltpu.VMEM((_CH, d), jnp.float32)
    scratch = [
        pltpu.VMEM((nch, _CH), jnp.int32),
        pltpu.VMEM((nch, _CH), jnp.int32),
        buf_shape,
        buf_shape,
        pltpu.VMEM_SHARED((half + 8, d), jnp.float32),
    ] + [pltpu.SemaphoreType.DMA] * 2

    @functools.partial(
        pl.kernel,
        mesh=mesh,
        out_type=jax.ShapeDtypeStruct((n, d), jnp.float32),
        scratch_types=scratch,
    )
    def k(table_hbm, aw_hbm, src_hbm, dst_hbm, x_hbm,
          out, srcb, dstb, gbuf, abuf, acc, gsem, asem):
        cid = lax.axis_index("c")
        sid = lax.axis_index("s")
        lo = cid * half
        r0 = sid * rpt

        pltpu.sync_copy(x_hbm.at[pl.ds(lo + r0, rpt)], acc.at[pl.ds(r0, rpt)])

        @pl.when(sid == ns - 1)
        def _init_tail():
            t = ns * rpt
            pltpu.sync_copy(x_hbm.at[pl.ds(lo + t, tail)],
                            acc.at[pl.ds(t, tail)])

        pltpu.sync_copy(src_hbm.at[sid], srcb)
        pltpu.sync_copy(dst_hbm.at[sid], dstb)

        @plsc.parallel_loop(0, nch, 1, unroll=2)
        def _clamp(c):
            for kk in range(_CH // 16):
                s = pl.ds(kk * 16, 16)
                dv = dstb[c, s]
                keep = (dv >= lo) & (dv < lo + half)
                dstb[c, s] = jnp.where(keep, dv - lo, trash)

        plsc.subcore_barrier()

        ebase = sid * ept

        @pl.loop(0, nch)
        def _(c):
            gd = pltpu.async_copy(table_hbm.at[srcb.at[c]], gbuf, gsem)
            if have_aw:
                ad = pltpu.async_copy(
                    aw_hbm.at[pl.ds(ebase + c * _CH, _CH)], abuf, asem)
            gd.wait()
            if have_aw:
                ad.wait()

                @plsc.parallel_loop(0, _CH, 1, unroll=2)
                def _(r):
                    for kk in range(d // 16):
                        s = pl.ds(kk * 16, 16)
                        abuf[r, s] = jnp.maximum(abuf[r, s] + gbuf[r, s], 0.0)

                pltpu.sync_copy(abuf, acc.at[dstb.at[c]], add=True)
            else:
                pltpu.sync_copy(gbuf, acc.at[dstb.at[c]], add=True)

        plsc.subcore_barrier()

        pltpu.sync_copy(acc.at[pl.ds(r0, rpt)], out.at[pl.ds(lo + r0, rpt)])

        @pl.when(sid == ns - 1)
        def _flush_tail():
            t = ns * rpt
            pltpu.sync_copy(acc.at[pl.ds(t, tail)],
                            out.at[pl.ds(lo + t, tail)])

    return k(table, aw if have_aw else table, src3, dst3, x)


def kernel(x, up_index, up_attr, face_index, face_attr,
           W_msg, b_msg, W1u, b1u, W2u, b2u, W1f, b1f, W2f, b2f, Wc, bc):
    n, d = x.shape
    e = up_attr.shape[0]

    xw = _mm(x, W_msg[:d])
    aw = _mm_bias(up_attr, W_msg[d:], b_msg, block=2000)

    ns = plsc.get_sparse_core_info().num_subcores
    idx_shape = (ns, e // ns // _CH, _CH)
    usrc3 = up_index[0].reshape(idx_shape)
    udst3 = up_index[1].reshape(idx_shape)
    fsrc3 = face_index[0].reshape(idx_shape)
    fdst3 = face_index[1].reshape(idx_shape)

    acc_up = _sc_task(xw, aw, x, usrc3, udst3)
    acc_f = _sc_task(face_attr, None, x, fsrc3, fdst3)

    return _mlps(acc_up, acc_f, W1u, b1u, W2u, b2u, W1f, b1f, W2f, b2f,
                 Wc, bc, block=1000)
